# baseline (speedup 1.0000x reference)
import functools

import jax
import jax.numpy as jnp
from jax import lax
from jax.experimental import pallas as pl
from jax.experimental.pallas import tpu as pltpu
from jax.experimental.pallas import tpu_sc as plsc

NC = 2
NS = 16
NW = NC * NS
CH = 128


def _sc_mesh():
  return plsc.VectorSubcoreMesh(
      core_axis_name="c", subcore_axis_name="s", num_cores=NC, num_subcores=NS)


def _make_deg(n, n_pad, e):
  epw = e // NW

  @functools.partial(
      pl.kernel,
      out_type=jax.ShapeDtypeStruct((NW * n_pad,), jnp.float32),
      mesh=_sc_mesh(),
      compiler_params=pltpu.CompilerParams(
          needs_layout_passes=False, use_tc_tiling_on_sc=False),
      scratch_types=[
          pltpu.VMEM((n_pad,), jnp.float32),
          pltpu.VMEM((epw,), jnp.int32),
      ],
  )
  def deg_kernel(ei_hbm, out_hbm, deg_v, dst_v):
    c = lax.axis_index("c")
    s = lax.axis_index("s")
    w = s * NC + c

    zeros16 = jnp.zeros((16,), jnp.float32)

    def zero_body(i, carry):
      deg_v[pl.ds(i * 16, 16)] = zeros16
      return carry

    lax.fori_loop(0, n_pad // 16, zero_body, 0)

    pltpu.sync_copy(ei_hbm.at[1, pl.ds(w * epw, epw)], dst_v)

    ones16 = jnp.ones((16,), jnp.float32)

    def body(i, carry):
      idx = dst_v[pl.ds(i * 16, 16)]
      plsc.addupdate_scatter(deg_v, [idx], ones16)
      return carry

    lax.fori_loop(0, epw // 16, body, 0)
    pltpu.sync_copy(deg_v, out_hbm.at[pl.ds(w * n_pad, n_pad)])

  return deg_kernel


def _make_agg(n, n_pad, e, d, nbuf):
  epw = e // NW
  nchunk = (epw + CH - 1) // CH
  while nchunk % nbuf:
    nchunk += 1
  epw_pad = nchunk * CH
  ngroup = nchunk // nbuf
  ntail = (epw_pad - epw) // 16
  assert (epw_pad - epw) % 16 == 0
  rows_per_tile = n_pad // NS

  @functools.partial(
      pl.kernel,
      out_type=jax.ShapeDtypeStruct((NC, n_pad, d), jnp.float32),
      mesh=_sc_mesh(),
      compiler_params=pltpu.CompilerParams(
          needs_layout_passes=False, use_tc_tiling_on_sc=False),
      scratch_types=[
          pltpu.VMEM((epw_pad,), jnp.int32),
          pltpu.VMEM((epw_pad,), jnp.int32),
          [pltpu.VMEM((CH, d), jnp.float32) for _ in range(nbuf)],
          pltpu.VMEM_SHARED((n_pad, d), jnp.float32),
          [pltpu.SemaphoreType.DMA for _ in range(nbuf)],
      ],
  )
  def agg_kernel(h_hbm, ei_hbm, zeros_hbm, out_hbm, src_v, dst_v, rows,
                 acc_sh, sems):
    c = lax.axis_index("c")
    s = lax.axis_index("s")
    w = s * NC + c
    r0 = s * rows_per_tile

    pltpu.sync_copy(ei_hbm.at[0, pl.ds(w * epw, epw)],
                    src_v.at[pl.ds(0, epw)])
    pltpu.sync_copy(ei_hbm.at[1, pl.ds(w * epw, epw)],
                    dst_v.at[pl.ds(0, epw)])
    zero16 = jnp.zeros((16,), jnp.int32)
    dead16 = jnp.full((16,), n, jnp.int32)
    for t in range(ntail):
      src_v[pl.ds(epw + t * 16, 16)] = zero16
      dst_v[pl.ds(epw + t * 16, 16)] = dead16

    pltpu.sync_copy(zeros_hbm.at[pl.ds(r0, rows_per_tile)],
                    acc_sh.at[pl.ds(r0, rows_per_tile)])
    plsc.subcore_barrier()

    for b in range(nbuf):
      pltpu.async_copy(h_hbm.at[src_v.at[pl.ds(b * CH, CH)]], rows[b],
                       sems[b])

    def group(g, carry):
      chunk0 = g * nbuf
      for b in range(nbuf):
        chunk = chunk0 + b
        pltpu.make_async_copy(h_hbm.at[src_v.at[pl.ds(chunk * CH, CH)]],
                              rows[b], sems[b]).wait()
        pltpu.sync_copy(rows[b], acc_sh.at[dst_v.at[pl.ds(chunk * CH, CH)]],
                        add=True)
        nxt = chunk + nbuf

        @pl.when(nxt < nchunk)
        def _():
          pltpu.async_copy(h_hbm.at[src_v.at[pl.ds(nxt * CH, CH)]], rows[b],
                           sems[b])

      return carry

    lax.fori_loop(0, ngroup, group, 0)

    plsc.subcore_barrier()
    pltpu.sync_copy(acc_sh.at[pl.ds(r0, rows_per_tile)],
                    out_hbm.at[c, pl.ds(r0, rows_per_tile)])

  return agg_kernel


def _dis_from_parts(degp_t):
  deg = jnp.sum(degp_t, axis=1)
  return jnp.where(deg > 0.0, lax.rsqrt(deg), 0.0)


def _make_h1s(n, d_in, d_h, blk):
  def body(x_ref, w_ref, degp_ref, o_ref):
    dis = _dis_from_parts(degp_ref[...])
    h = jnp.dot(x_ref[...], w_ref[...], preferred_element_type=jnp.float32)
    o_ref[...] = h * dis[:, None]

  return pl.pallas_call(
      body,
      grid=(n // blk,),
      in_specs=[
          pl.BlockSpec((blk, d_in), lambda i: (i, 0)),
          pl.BlockSpec((d_in, d_h), lambda i: (0, 0)),
          pl.BlockSpec((blk, NW), lambda i: (i, 0)),
      ],
      out_specs=pl.BlockSpec((blk, d_h), lambda i: (i, 0)),
      out_shape=jax.ShapeDtypeStruct((n, d_h), jnp.float32),
  )


def _make_h2s(n_pad, d_h, d_out, blk):
  def body(agg_ref, degp_ref, b_ref, w_ref, o_ref):
    dis = _dis_from_parts(degp_ref[...])
    a = agg_ref[...]
    t = (a[0] + a[1]) * dis[:, None] + b_ref[...]
    h1 = jnp.maximum(t, 0.0)
    h2 = jnp.dot(h1, w_ref[...], preferred_element_type=jnp.float32)
    o_ref[...] = h2 * dis[:, None]

  return pl.pallas_call(
      body,
      grid=(n_pad // blk,),
      in_specs=[
          pl.BlockSpec((NC, blk, d_h), lambda i: (0, i, 0)),
          pl.BlockSpec((blk, NW), lambda i: (i, 0)),
          pl.BlockSpec((1, d_h), lambda i: (0, 0)),
          pl.BlockSpec((d_h, d_out), lambda i: (0, 0)),
      ],
      out_specs=pl.BlockSpec((blk, d_out), lambda i: (i, 0)),
      out_shape=jax.ShapeDtypeStruct((n_pad, d_out), jnp.float32),
  )


def _make_softmax_out(n, d_out, blk):
  def body(agg_ref, degp_ref, b_ref, o_ref):
    dis = _dis_from_parts(degp_ref[...])
    a = agg_ref[...]
    t = (a[0] + a[1]) * dis[:, None] + b_ref[...]
    m = jnp.max(t, axis=1, keepdims=True)
    ex = jnp.exp(t - m)
    o_ref[...] = ex / jnp.sum(ex, axis=1, keepdims=True)

  return pl.pallas_call(
      body,
      grid=(n // blk,),
      in_specs=[
          pl.BlockSpec((NC, blk, d_out), lambda i: (0, i, 0)),
          pl.BlockSpec((blk, NW), lambda i: (i, 0)),
          pl.BlockSpec((1, d_out), lambda i: (0, 0)),
      ],
      out_specs=pl.BlockSpec((blk, d_out), lambda i: (i, 0)),
      out_shape=jax.ShapeDtypeStruct((n, d_out), jnp.float32),
  )


def kernel(x, edge_indices, W1, b1, W2, b2):
  n, d_in = x.shape
  e = edge_indices.shape[1]
  d_h = W1.shape[1]
  d_out = W2.shape[1]
  blk = 1000

  n_pad = ((n + 8 * NS - 1) // (8 * NS)) * (8 * NS)
  blk2 = n_pad // 8

  ei = edge_indices.astype(jnp.int32)
  zeros_h = jnp.zeros((n_pad, d_h), jnp.float32)

  deg_parts = _make_deg(n, n_pad, e)(ei)
  degp_t = deg_parts.reshape(NW, n_pad).T
  agg = _make_agg(n, n_pad, e, d_h, 4)
  h1s = _make_h1s(n, d_in, d_h, blk)(x, W1, degp_t)
  agg1 = agg(h1s, ei, zeros_h)
  h2s = _make_h2s(n_pad, d_h, d_out, blk2)(agg1, degp_t, b1.reshape(1, d_h),
                                           W2)
  agg2 = agg(h2s, ei, zeros_h)
  return _make_softmax_out(n, d_out, blk)(agg2, degp_t, b2.reshape(1, d_out))


# speedup vs baseline: 2.2757x; 2.2757x over previous
import functools

import jax
import jax.numpy as jnp
from jax import lax
from jax.experimental import pallas as pl
from jax.experimental.pallas import tpu as pltpu
from jax.experimental.pallas import tpu_sc as plsc

NC = 2
NS = 16
NW = NC * NS
CH = 128


def _sc_mesh():
  return plsc.VectorSubcoreMesh(
      core_axis_name="c", subcore_axis_name="s", num_cores=NC, num_subcores=NS)


def _make_deg(n, n_pad, e):
  epw = e // NW

  @functools.partial(
      pl.kernel,
      out_type=jax.ShapeDtypeStruct((NW * n_pad,), jnp.float32),
      mesh=_sc_mesh(),
      compiler_params=pltpu.CompilerParams(
          needs_layout_passes=False, use_tc_tiling_on_sc=False),
      scratch_types=[
          pltpu.VMEM((n_pad,), jnp.float32),
          pltpu.VMEM((epw,), jnp.int32),
      ],
  )
  def deg_kernel(ei_hbm, out_hbm, deg_v, dst_v):
    c = lax.axis_index("c")
    s = lax.axis_index("s")
    w = s * NC + c

    zeros16 = jnp.zeros((16,), jnp.float32)

    def zero_body(i, carry):
      deg_v[pl.ds(i * 16, 16)] = zeros16
      return carry

    lax.fori_loop(0, n_pad // 16, zero_body, 0)

    pltpu.sync_copy(ei_hbm.at[1, pl.ds(w * epw, epw)], dst_v)

    ones16 = jnp.ones((16,), jnp.float32)

    def body(i, carry):
      idx = dst_v[pl.ds(i * 16, 16)]
      plsc.addupdate_scatter(deg_v, [idx], ones16)
      return carry

    lax.fori_loop(0, epw // 16, body, 0)
    pltpu.sync_copy(deg_v, out_hbm.at[pl.ds(w * n_pad, n_pad)])

  return deg_kernel


def _make_agg(n, n_pad, e, d, ch, nbuf):
  epw = e // NW
  nchunk = epw // ch
  ngroup = nchunk // nbuf
  rows_per_tile = n_pad // NS

  @functools.partial(
      pl.kernel,
      out_type=jax.ShapeDtypeStruct((NC, n_pad, d), jnp.float32),
      mesh=_sc_mesh(),
      compiler_params=pltpu.CompilerParams(
          needs_layout_passes=False, use_tc_tiling_on_sc=False),
      scratch_types=[
          pltpu.VMEM((nchunk, ch), jnp.int32),
          pltpu.VMEM((nchunk, ch), jnp.int32),
          [pltpu.VMEM((ch, d), jnp.float32) for _ in range(nbuf)],
          pltpu.VMEM_SHARED((n_pad, d), jnp.float32),
          [pltpu.SemaphoreType.DMA for _ in range(nbuf)],
      ],
  )
  def agg_kernel(h_hbm, src_hbm, dst_hbm, zeros_hbm, out_hbm, src_v, dst_v,
                 rows, acc_sh, sems):
    c = lax.axis_index("c")
    s = lax.axis_index("s")
    w = s * NC + c
    r0 = s * rows_per_tile

    pltpu.sync_copy(src_hbm.at[w], src_v)
    pltpu.sync_copy(dst_hbm.at[w], dst_v)

    pltpu.sync_copy(zeros_hbm.at[pl.ds(r0, rows_per_tile)],
                    acc_sh.at[pl.ds(r0, rows_per_tile)])
    plsc.subcore_barrier()

    for b in range(nbuf):
      pltpu.async_copy(h_hbm.at[src_v.at[b]], rows[b], sems[b])

    def group(g, carry):
      chunk0 = g * nbuf
      for b in range(nbuf):
        chunk = chunk0 + b
        pltpu.make_async_copy(h_hbm.at[src_v.at[chunk]], rows[b],
                              sems[b]).wait()
        pltpu.sync_copy(rows[b], acc_sh.at[dst_v.at[chunk]], add=True)
        nxt = chunk + nbuf

        @pl.when(nxt < nchunk)
        def _():
          pltpu.async_copy(h_hbm.at[src_v.at[nxt]], rows[b], sems[b])

      return carry

    lax.fori_loop(0, ngroup, group, 0)

    plsc.subcore_barrier()
    pltpu.sync_copy(acc_sh.at[pl.ds(r0, rows_per_tile)],
                    out_hbm.at[c, pl.ds(r0, rows_per_tile)])

  return agg_kernel


def _dis_from_parts(degp_t):
  deg = jnp.sum(degp_t, axis=1)
  return jnp.where(deg > 0.0, lax.rsqrt(deg), 0.0)


def _make_h1s(n, d_in, d_h, blk):
  def body(x_ref, w_ref, degp_ref, o_ref):
    dis = _dis_from_parts(degp_ref[...])
    h = jnp.dot(x_ref[...], w_ref[...], preferred_element_type=jnp.float32)
    o_ref[...] = h * dis[:, None]

  return pl.pallas_call(
      body,
      grid=(n // blk,),
      in_specs=[
          pl.BlockSpec((blk, d_in), lambda i: (i, 0)),
          pl.BlockSpec((d_in, d_h), lambda i: (0, 0)),
          pl.BlockSpec((blk, NW), lambda i: (i, 0)),
      ],
      out_specs=pl.BlockSpec((blk, d_h), lambda i: (i, 0)),
      out_shape=jax.ShapeDtypeStruct((n, d_h), jnp.float32),
  )


def _make_h2s(n_pad, d_h, d_out, blk):
  def body(agg_ref, degp_ref, b_ref, w_ref, o_ref):
    dis = _dis_from_parts(degp_ref[...])
    a = agg_ref[...]
    t = (a[0] + a[1]) * dis[:, None] + b_ref[...]
    h1 = jnp.maximum(t, 0.0)
    h2 = jnp.dot(h1, w_ref[...], preferred_element_type=jnp.float32)
    o_ref[...] = h2 * dis[:, None]

  return pl.pallas_call(
      body,
      grid=(n_pad // blk,),
      in_specs=[
          pl.BlockSpec((NC, blk, d_h), lambda i: (0, i, 0)),
          pl.BlockSpec((blk, NW), lambda i: (i, 0)),
          pl.BlockSpec((1, d_h), lambda i: (0, 0)),
          pl.BlockSpec((d_h, d_out), lambda i: (0, 0)),
      ],
      out_specs=pl.BlockSpec((blk, d_out), lambda i: (i, 0)),
      out_shape=jax.ShapeDtypeStruct((n_pad, d_out), jnp.float32),
  )


def _make_softmax_out(n, d_out, blk):
  def body(agg_ref, degp_ref, b_ref, o_ref):
    dis = _dis_from_parts(degp_ref[...])
    a = agg_ref[...]
    t = (a[0] + a[1]) * dis[:, None] + b_ref[...]
    m = jnp.max(t, axis=1, keepdims=True)
    ex = jnp.exp(t - m)
    o_ref[...] = ex / jnp.sum(ex, axis=1, keepdims=True)

  return pl.pallas_call(
      body,
      grid=(n // blk,),
      in_specs=[
          pl.BlockSpec((NC, blk, d_out), lambda i: (0, i, 0)),
          pl.BlockSpec((blk, NW), lambda i: (i, 0)),
          pl.BlockSpec((1, d_out), lambda i: (0, 0)),
      ],
      out_specs=pl.BlockSpec((blk, d_out), lambda i: (i, 0)),
      out_shape=jax.ShapeDtypeStruct((n, d_out), jnp.float32),
  )


def kernel(x, edge_indices, W1, b1, W2, b2):
  n, d_in = x.shape
  e = edge_indices.shape[1]
  d_h = W1.shape[1]
  d_out = W2.shape[1]
  blk = 1000

  n_pad = ((n + 8 * NS - 1) // (8 * NS)) * (8 * NS)
  blk2 = n_pad // 8

  ch = 125
  epw = e // NW
  nchunk = epw // ch

  ei = edge_indices.astype(jnp.int32)
  src3 = ei[0].reshape(NW, nchunk, ch)
  dst3 = ei[1].reshape(NW, nchunk, ch)
  zeros_h = jnp.zeros((n_pad, d_h), jnp.float32)

  deg_parts = _make_deg(n, n_pad, e)(ei)
  degp_t = deg_parts.reshape(NW, n_pad).T
  agg = _make_agg(n, n_pad, e, d_h, ch, 4)
  h1s = _make_h1s(n, d_in, d_h, blk)(x, W1, degp_t)
  agg1 = agg(h1s, src3, dst3, zeros_h)
  h2s = _make_h2s(n_pad, d_h, d_out, blk2)(agg1, degp_t, b1.reshape(1, d_h),
                                           W2)
  agg2 = agg(h2s, src3, dst3, zeros_h)
  return _make_softmax_out(n, d_out, blk)(agg2, degp_t, b2.reshape(1, d_out))


# speedup vs baseline: 2.3808x; 1.0462x over previous
import functools

import jax
import jax.numpy as jnp
from jax import lax
from jax.experimental import pallas as pl
from jax.experimental.pallas import tpu as pltpu
from jax.experimental.pallas import tpu_sc as plsc

NC = 2
NS = 16
NW = NC * NS
CH = 128


def _sc_mesh():
  return plsc.VectorSubcoreMesh(
      core_axis_name="c", subcore_axis_name="s", num_cores=NC, num_subcores=NS)


def _make_deg(n, n_pad, e):
  epw = e // NW

  @functools.partial(
      pl.kernel,
      out_type=jax.ShapeDtypeStruct((NW * n_pad,), jnp.float32),
      mesh=_sc_mesh(),
      compiler_params=pltpu.CompilerParams(
          needs_layout_passes=False, use_tc_tiling_on_sc=False),
      scratch_types=[
          pltpu.VMEM((n_pad,), jnp.float32),
          pltpu.VMEM((epw,), jnp.int32),
      ],
  )
  def deg_kernel(ei_hbm, out_hbm, deg_v, dst_v):
    c = lax.axis_index("c")
    s = lax.axis_index("s")
    w = s * NC + c

    zeros16 = jnp.zeros((16,), jnp.float32)

    def zero_body(i, carry):
      deg_v[pl.ds(i * 16, 16)] = zeros16
      return carry

    lax.fori_loop(0, n_pad // 16, zero_body, 0)

    pltpu.sync_copy(ei_hbm.at[1, pl.ds(w * epw, epw)], dst_v)

    ones16 = jnp.ones((16,), jnp.float32)

    def body(i, carry):
      idx = dst_v[pl.ds(i * 16, 16)]
      plsc.addupdate_scatter(deg_v, [idx], ones16)
      return carry

    lax.fori_loop(0, epw // 16, body, 0)
    pltpu.sync_copy(deg_v, out_hbm.at[pl.ds(w * n_pad, n_pad)])

  return deg_kernel


def _make_agg(n, n_pad, e, d, ch, nbuf):
  epw = e // NW
  nchunk = epw // ch
  ngroup = nchunk // nbuf
  rows_per_tile = n_pad // NS

  @functools.partial(
      pl.kernel,
      out_type=jax.ShapeDtypeStruct((NC, n_pad, d), jnp.float32),
      mesh=_sc_mesh(),
      compiler_params=pltpu.CompilerParams(
          needs_layout_passes=False, use_tc_tiling_on_sc=False),
      scratch_types=[
          pltpu.VMEM((nchunk, ch), jnp.int32),
          pltpu.VMEM((nchunk, ch), jnp.int32),
          [pltpu.VMEM((ch, d), jnp.float32) for _ in range(nbuf)],
          pltpu.VMEM_SHARED((n_pad, d), jnp.float32),
          [pltpu.SemaphoreType.DMA for _ in range(nbuf)],
      ],
  )
  def agg_kernel(h_hbm, ei4_hbm, zeros_hbm, out_hbm, src_v, dst_v,
                 rows, acc_sh, sems):
    c = lax.axis_index("c")
    s = lax.axis_index("s")
    w = s * NC + c
    r0 = s * rows_per_tile

    pltpu.sync_copy(ei4_hbm.at[0, w], src_v)
    pltpu.sync_copy(ei4_hbm.at[1, w], dst_v)

    pltpu.sync_copy(zeros_hbm.at[pl.ds(r0, rows_per_tile)],
                    acc_sh.at[pl.ds(r0, rows_per_tile)])
    plsc.subcore_barrier()

    for b in range(nbuf):
      pltpu.async_copy(h_hbm.at[src_v.at[b]], rows[b], sems[b])

    def group(g, carry):
      chunk0 = g * nbuf
      for b in range(nbuf):
        chunk = chunk0 + b
        pltpu.make_async_copy(h_hbm.at[src_v.at[chunk]], rows[b],
                              sems[b]).wait()
        pltpu.sync_copy(rows[b], acc_sh.at[dst_v.at[chunk]], add=True)
        nxt = chunk + nbuf

        @pl.when(nxt < nchunk)
        def _():
          pltpu.async_copy(h_hbm.at[src_v.at[nxt]], rows[b], sems[b])

      return carry

    lax.fori_loop(0, ngroup, group, 0)

    plsc.subcore_barrier()
    pltpu.sync_copy(acc_sh.at[pl.ds(r0, rows_per_tile)],
                    out_hbm.at[c, pl.ds(r0, rows_per_tile)])

  return agg_kernel


def _dis_from_parts(degp_t):
  deg = jnp.sum(degp_t, axis=1)
  return jnp.where(deg > 0.0, lax.rsqrt(deg), 0.0)


def _make_h1s(n, d_in, d_h, blk):
  def body(x_ref, w_ref, degp_ref, o_ref):
    dis = _dis_from_parts(degp_ref[...])
    h = jnp.dot(x_ref[...], w_ref[...], preferred_element_type=jnp.float32)
    o_ref[...] = h * dis[:, None]

  return pl.pallas_call(
      body,
      grid=(n // blk,),
      in_specs=[
          pl.BlockSpec((blk, d_in), lambda i: (i, 0)),
          pl.BlockSpec((d_in, d_h), lambda i: (0, 0)),
          pl.BlockSpec((blk, NW), lambda i: (i, 0)),
      ],
      out_specs=pl.BlockSpec((blk, d_h), lambda i: (i, 0)),
      out_shape=jax.ShapeDtypeStruct((n, d_h), jnp.float32),
  )


def _make_h2s(n_pad, d_h, d_out, blk):
  def body(agg_ref, degp_ref, b_ref, w_ref, o_ref):
    dis = _dis_from_parts(degp_ref[...])
    a = agg_ref[...]
    t = (a[0] + a[1]) * dis[:, None] + b_ref[...]
    h1 = jnp.maximum(t, 0.0)
    h2 = jnp.dot(h1, w_ref[...], preferred_element_type=jnp.float32)
    o_ref[...] = h2 * dis[:, None]

  return pl.pallas_call(
      body,
      grid=(n_pad // blk,),
      in_specs=[
          pl.BlockSpec((NC, blk, d_h), lambda i: (0, i, 0)),
          pl.BlockSpec((blk, NW), lambda i: (i, 0)),
          pl.BlockSpec((1, d_h), lambda i: (0, 0)),
          pl.BlockSpec((d_h, d_out), lambda i: (0, 0)),
      ],
      out_specs=pl.BlockSpec((blk, d_out), lambda i: (i, 0)),
      out_shape=jax.ShapeDtypeStruct((n_pad, d_out), jnp.float32),
  )


def _make_softmax_out(n, d_out, blk):
  def body(agg_ref, degp_ref, b_ref, o_ref):
    dis = _dis_from_parts(degp_ref[...])
    a = agg_ref[...]
    t = (a[0] + a[1]) * dis[:, None] + b_ref[...]
    m = jnp.max(t, axis=1, keepdims=True)
    ex = jnp.exp(t - m)
    o_ref[...] = ex / jnp.sum(ex, axis=1, keepdims=True)

  return pl.pallas_call(
      body,
      grid=(n // blk,),
      in_specs=[
          pl.BlockSpec((NC, blk, d_out), lambda i: (0, i, 0)),
          pl.BlockSpec((blk, NW), lambda i: (i, 0)),
          pl.BlockSpec((1, d_out), lambda i: (0, 0)),
      ],
      out_specs=pl.BlockSpec((blk, d_out), lambda i: (i, 0)),
      out_shape=jax.ShapeDtypeStruct((n, d_out), jnp.float32),
  )


def kernel(x, edge_indices, W1, b1, W2, b2):
  n, d_in = x.shape
  e = edge_indices.shape[1]
  d_h = W1.shape[1]
  d_out = W2.shape[1]
  blk = 1000

  n_pad = ((n + 8 * NS - 1) // (8 * NS)) * (8 * NS)
  blk2 = n_pad // 8

  ch = 125
  epw = e // NW
  nchunk = epw // ch

  ei = edge_indices.astype(jnp.int32)
  ei4 = ei.reshape(2, NW, nchunk, ch)
  zeros_h = jnp.zeros((n_pad, d_h), jnp.float32)

  deg_parts = _make_deg(n, n_pad, e)(ei)
  degp_t = deg_parts.reshape(NW, n_pad).T
  agg = _make_agg(n, n_pad, e, d_h, ch, 8)
  h1s = _make_h1s(n, d_in, d_h, blk)(x, W1, degp_t)
  agg1 = agg(h1s, ei4, zeros_h)
  h2s = _make_h2s(n_pad, d_h, d_out, blk2)(agg1, degp_t, b1.reshape(1, d_h),
                                           W2)
  agg2 = agg(h2s, ei4, zeros_h)
  return _make_softmax_out(n, d_out, blk)(agg2, degp_t, b2.reshape(1, d_out))


# speedup vs baseline: 2.4551x; 1.0312x over previous
import functools

import jax
import jax.numpy as jnp
from jax import lax
from jax.experimental import pallas as pl
from jax.experimental.pallas import tpu as pltpu
from jax.experimental.pallas import tpu_sc as plsc

NC = 2
NS = 16
NW = NC * NS
CH = 128


def _sc_mesh():
  return plsc.VectorSubcoreMesh(
      core_axis_name="c", subcore_axis_name="s", num_cores=NC, num_subcores=NS)


def _make_deg(n, n_pad, e):
  epw = e // NW

  @functools.partial(
      pl.kernel,
      out_type=jax.ShapeDtypeStruct((NW * n_pad,), jnp.float32),
      mesh=_sc_mesh(),
      compiler_params=pltpu.CompilerParams(
          needs_layout_passes=False, use_tc_tiling_on_sc=False),
      scratch_types=[
          pltpu.VMEM((n_pad,), jnp.float32),
          pltpu.VMEM((epw,), jnp.int32),
      ],
  )
  def deg_kernel(ei_hbm, out_hbm, deg_v, dst_v):
    c = lax.axis_index("c")
    s = lax.axis_index("s")
    w = s * NC + c

    zeros16 = jnp.zeros((16,), jnp.float32)

    def zero_body(i, carry):
      deg_v[pl.ds(i * 16, 16)] = zeros16
      return carry

    lax.fori_loop(0, n_pad // 16, zero_body, 0)

    pltpu.sync_copy(ei_hbm.at[1, pl.ds(w * epw, epw)], dst_v)

    ones16 = jnp.ones((16,), jnp.float32)

    def body(i, carry):
      idx = dst_v[pl.ds(i * 16, 16)]
      plsc.addupdate_scatter(deg_v, [idx], ones16)
      return carry

    lax.fori_loop(0, epw // 16, body, 0)
    pltpu.sync_copy(deg_v, out_hbm.at[pl.ds(w * n_pad, n_pad)])

  return deg_kernel


def _make_agg(n, n_pad, e, d, ch, nbuf):
  epw = e // NW
  nchunk = epw // ch
  ngroup = nchunk // nbuf
  rows_per_tile = n_pad // NS

  @functools.partial(
      pl.kernel,
      out_type=jax.ShapeDtypeStruct((NC, n_pad, d), jnp.float32),
      mesh=_sc_mesh(),
      compiler_params=pltpu.CompilerParams(
          needs_layout_passes=False, use_tc_tiling_on_sc=False),
      scratch_types=[
          pltpu.VMEM((nchunk, ch), jnp.int32),
          pltpu.VMEM((nchunk, ch), jnp.int32),
          [pltpu.VMEM((ch, d), jnp.float32) for _ in range(nbuf)],
          pltpu.VMEM_SHARED((n_pad, d), jnp.float32),
          [pltpu.SemaphoreType.DMA for _ in range(nbuf)],
      ],
  )
  def agg_kernel(h_hbm, ei4_hbm, zeros_hbm, out_hbm, src_v, dst_v,
                 rows, acc_sh, sems):
    c = lax.axis_index("c")
    s = lax.axis_index("s")
    w = s * NC + c
    r0 = s * rows_per_tile

    pltpu.sync_copy(ei4_hbm.at[0, w], src_v)
    pltpu.sync_copy(ei4_hbm.at[1, w], dst_v)

    pltpu.sync_copy(zeros_hbm.at[pl.ds(r0, rows_per_tile)],
                    acc_sh.at[pl.ds(r0, rows_per_tile)])
    plsc.subcore_barrier()

    for b in range(nbuf):
      pltpu.async_copy(h_hbm.at[src_v.at[b]], rows[b], sems[b])

    def group(g, carry):
      chunk0 = g * nbuf
      for b in range(nbuf):
        chunk = chunk0 + b
        pltpu.make_async_copy(h_hbm.at[src_v.at[chunk]], rows[b],
                              sems[b]).wait()
        pltpu.sync_copy(rows[b], acc_sh.at[dst_v.at[chunk]], add=True)
        nxt = chunk + nbuf

        @pl.when(nxt < nchunk)
        def _():
          pltpu.async_copy(h_hbm.at[src_v.at[nxt]], rows[b], sems[b])

      return carry

    lax.fori_loop(0, ngroup, group, 0)

    plsc.subcore_barrier()
    pltpu.sync_copy(acc_sh.at[pl.ds(r0, rows_per_tile)],
                    out_hbm.at[c, pl.ds(r0, rows_per_tile)])

  return agg_kernel


def _dis_from_parts(degp_t):
  deg = jnp.sum(degp_t, axis=1)
  return jnp.where(deg > 0.0, lax.rsqrt(deg), 0.0)


def _make_h1s(n, d_in, d_h, blk):
  def body(x_ref, w_ref, degp_ref, o_ref):
    dis = _dis_from_parts(degp_ref[...])
    h = jnp.dot(x_ref[...], w_ref[...], preferred_element_type=jnp.float32)
    o_ref[...] = h * dis[:, None]

  return pl.pallas_call(
      body,
      grid=(n // blk,),
      in_specs=[
          pl.BlockSpec((blk, d_in), lambda i: (i, 0)),
          pl.BlockSpec((d_in, d_h), lambda i: (0, 0)),
          pl.BlockSpec((blk, NW), lambda i: (i, 0)),
      ],
      out_specs=pl.BlockSpec((blk, d_h), lambda i: (i, 0)),
      out_shape=jax.ShapeDtypeStruct((n, d_h), jnp.float32),
  )


def _make_h2s(n_pad, d_h, d_out, blk):
  def body(agg_ref, degp_ref, b_ref, w_ref, o_ref):
    dis = _dis_from_parts(degp_ref[...])
    a = agg_ref[...]
    t = (a[0] + a[1]) * dis[:, None] + b_ref[...]
    h1 = jnp.maximum(t, 0.0)
    h2 = jnp.dot(h1, w_ref[...], preferred_element_type=jnp.float32)
    o_ref[...] = h2 * dis[:, None]

  return pl.pallas_call(
      body,
      grid=(n_pad // blk,),
      in_specs=[
          pl.BlockSpec((NC, blk, d_h), lambda i: (0, i, 0)),
          pl.BlockSpec((blk, NW), lambda i: (i, 0)),
          pl.BlockSpec((1, d_h), lambda i: (0, 0)),
          pl.BlockSpec((d_h, d_out), lambda i: (0, 0)),
      ],
      out_specs=pl.BlockSpec((blk, d_out), lambda i: (i, 0)),
      out_shape=jax.ShapeDtypeStruct((n_pad, d_out), jnp.float32),
  )


def _make_softmax_out(n, d_out, blk):
  def body(agg_ref, degp_ref, b_ref, o_ref):
    dis = _dis_from_parts(degp_ref[...])
    a = agg_ref[...]
    t = (a[0] + a[1]) * dis[:, None] + b_ref[...]
    m = jnp.max(t, axis=1, keepdims=True)
    ex = jnp.exp(t - m)
    o_ref[...] = ex / jnp.sum(ex, axis=1, keepdims=True)

  return pl.pallas_call(
      body,
      grid=(n // blk,),
      in_specs=[
          pl.BlockSpec((NC, blk, d_out), lambda i: (0, i, 0)),
          pl.BlockSpec((blk, NW), lambda i: (i, 0)),
          pl.BlockSpec((1, d_out), lambda i: (0, 0)),
      ],
      out_specs=pl.BlockSpec((blk, d_out), lambda i: (i, 0)),
      out_shape=jax.ShapeDtypeStruct((n, d_out), jnp.float32),
  )


def kernel(x, edge_indices, W1, b1, W2, b2):
  n, d_in = x.shape
  e = edge_indices.shape[1]
  d_h = W1.shape[1]
  d_out = W2.shape[1]
  blk = 2000

  n_pad = ((n + 8 * NS - 1) // (8 * NS)) * (8 * NS)
  blk2 = n_pad // 4

  ch = 125
  epw = e // NW
  nchunk = epw // ch

  ei = edge_indices.astype(jnp.int32)
  ei4 = ei.reshape(2, NW, nchunk, ch)
  zeros_h = jnp.zeros((n_pad, d_h), jnp.float32)

  deg_parts = _make_deg(n, n_pad, e)(ei)
  degp_t = deg_parts.reshape(NW, n_pad).T
  agg = _make_agg(n, n_pad, e, d_h, ch, 8)
  h1s = _make_h1s(n, d_in, d_h, blk)(x, W1, degp_t)
  agg1 = agg(h1s, ei4, zeros_h)
  h2s = _make_h2s(n_pad, d_h, d_out, blk2)(agg1, degp_t, b1.reshape(1, d_h),
                                           W2)
  agg2 = agg(h2s, ei4, zeros_h)
  return _make_softmax_out(n, d_out, blk)(agg2, degp_t, b2.reshape(1, d_out))
